# Initial kernel scaffold; baseline (speedup 1.0000x reference)
import jax
import jax.numpy as jnp
from jax.experimental import pallas as pl


def kernel(image):
    raise NotImplementedError("write your pallas kernel here")


# speedup vs baseline: 313.6567x; 313.6567x over previous
import functools

import jax
import jax.numpy as jnp
from jax import lax
from jax.experimental import pallas as pl
from jax.experimental.pallas import tpu as pltpu
from jax.experimental.pallas import tpu_sc as plsc

NCORES = 2
NSUB = 16
LANES = 16
NPIX = 512 * 512
CHUNK = NPIX // NSUB
NCH_PER_CORE = 24
NBINS = 256


def _floorf(x):
    return x.astype(jnp.int32).astype(jnp.float32)


def _equalize_body(img, out, pix, hist, histred, cum, lut, outb,
                   hist_sh, lut_sh):
    c = lax.axis_index("c")
    s = lax.axis_index("s")
    iota = lax.iota(jnp.int32, LANES)
    lane_base = iota * NBINS
    ones = jnp.ones((LANES,), jnp.float32)
    zeros = jnp.zeros((LANES,), jnp.float32)

    def p1_body(ch, _):
        base = (c * NCH_PER_CORE + ch) * NPIX + s * CHUNK
        pltpu.sync_copy(img.at[pl.ds(base, CHUNK)], pix)

        def zbody(j, _):
            for r in range(NSUB):
                hist[r, pl.ds(j * LANES, LANES)] = zeros
            return 0
        lax.fori_loop(0, NBINS // LANES, zbody, 0)

        def sbody(i, _):
            off = i * 256
            for u in range(16):
                v = pix[pl.ds(off + u * LANES, LANES)]
                plsc.addupdate_scatter(hist, [iota, v], ones)
            return 0
        lax.fori_loop(0, CHUNK // 256, sbody, 0)

        def rbody(j, _):
            acc = hist[0, pl.ds(j * LANES, LANES)]
            for r in range(1, LANES):
                acc = acc + hist[r, pl.ds(j * LANES, LANES)]
            histred[pl.ds(j * LANES, LANES)] = acc
            return 0
        lax.fori_loop(0, NBINS // LANES, rbody, 0)

        pltpu.sync_copy(histred, hist_sh.at[ch, s])
        return 0
    lax.fori_loop(0, NCH_PER_CORE, p1_body, 0)

    plsc.subcore_barrier()

    def make_lut(chv):
        pltpu.sync_copy(hist_sh.at[chv], hist)

        def rbody(j, _):
            acc = hist[0, pl.ds(j * LANES, LANES)]
            for r in range(1, NSUB):
                acc = acc + hist[r, pl.ds(j * LANES, LANES)]
            histred[pl.ds(j * LANES, LANES)] = acc
            return 0
        lax.fori_loop(0, NBINS // LANES, rbody, 0)

        def cbody(j, carry):
            cacc, li = carry
            x = histred[pl.ds(j * LANES, LANES)]
            cs = plsc.cumsum(x) + cacc
            cum[pl.ds(j * LANES, LANES)] = cs
            gidx = iota + j * LANES
            ljm = jnp.max(jnp.where(x > 0.0, gidx, -1))
            return (jnp.max(cs), jnp.maximum(li, ljm))
        total, li = lax.fori_loop(
            0, NBINS // LANES, cbody, (jnp.float32(0.0), jnp.int32(-1)))

        def hbody(j, acc):
            x = histred[pl.ds(j * LANES, LANES)]
            gidx = iota + j * LANES
            return acc + jnp.sum(jnp.where(gidx == li, x, 0.0))
        hist_last = lax.fori_loop(0, NBINS // LANES, hbody, jnp.float32(0.0))

        num_v = jnp.full((LANES,), total - hist_last, jnp.float32)
        step = _floorf(num_v / 255.0)
        half = _floorf(step * 0.5)
        div = jnp.maximum(step, 1.0)
        ident = step <= 0.0

        def lbody(j, _):
            cs = cum[pl.ds(j * LANES, LANES)]
            val = jnp.clip(_floorf((cs + half) / div), 0.0, 255.0)
            gidx = iota + j * LANES
            val = jnp.where(ident, (gidx + 1).astype(jnp.float32), val)
            plsc.store_scatter(lut, [gidx + 1], val, mask=gidx < NBINS - 1)
            return 0
        lax.fori_loop(0, NBINS // LANES, lbody, 0)
        v0 = lut[pl.ds(0, LANES)]
        lut[pl.ds(0, LANES)] = jnp.where(iota == 0, 0.0, v0)

        pltpu.sync_copy(lut, lut_sh.at[chv])

    for rep in range(2):
        chv = s + NSUB * rep

        @pl.when(chv < NCH_PER_CORE)
        def _(chv=chv):
            make_lut(chv)

    plsc.subcore_barrier()

    def p2_body(ch, _):
        base = (c * NCH_PER_CORE + ch) * NPIX + s * CHUNK
        pltpu.sync_copy(lut_sh.at[ch], lut)
        pltpu.sync_copy(img.at[pl.ds(base, CHUNK)], pix)

        def gbody(i, _):
            off = i * 256
            for u in range(16):
                sl = pl.ds(off + u * LANES, LANES)
                outb[sl] = plsc.load_gather(lut, [pix[sl]])
            return 0
        lax.fori_loop(0, CHUNK // 256, gbody, 0)

        pltpu.sync_copy(outb, out.at[pl.ds(base, CHUNK)])
        return 0
    lax.fori_loop(0, NCH_PER_CORE, p2_body, 0)


@jax.jit
def kernel(image):
    B, C, H, W = image.shape
    n = B * C * H * W
    flat = image.reshape(n)

    mesh = plsc.VectorSubcoreMesh(
        core_axis_name="c", subcore_axis_name="s",
        num_cores=NCORES, num_subcores=NSUB)
    eq = pl.kernel(
        _equalize_body,
        out_type=jax.ShapeDtypeStruct((n,), jnp.float32),
        mesh=mesh,
        compiler_params=pltpu.CompilerParams(
            use_tc_tiling_on_sc=False, needs_layout_passes=False),
        scratch_types=[
            pltpu.VMEM((CHUNK,), jnp.int32),
            pltpu.VMEM((NSUB, NBINS), jnp.float32),
            pltpu.VMEM((NBINS,), jnp.float32),
            pltpu.VMEM((NBINS,), jnp.float32),
            pltpu.VMEM((NBINS,), jnp.float32),
            pltpu.VMEM((CHUNK,), jnp.float32),
            pltpu.VMEM_SHARED((NCH_PER_CORE, NSUB, NBINS), jnp.float32),
            pltpu.VMEM_SHARED((NCH_PER_CORE, NBINS), jnp.float32),
        ],
    )
    return eq(flat).reshape(B, C, H, W)


# speedup vs baseline: 530.7169x; 1.6920x over previous
import functools

import jax
import jax.numpy as jnp
from jax import lax
from jax.experimental import pallas as pl
from jax.experimental.pallas import tpu as pltpu
from jax.experimental.pallas import tpu_sc as plsc

NCORES = 2
NSUB = 16
LANES = 16
NPIX = 512 * 512
CHUNK = NPIX // NSUB
NCH_PER_CORE = 24
NBINS = 256


def _floorf(x):
    return x.astype(jnp.int32).astype(jnp.float32)


def _equalize_body(img, out, pix, hist, histred, cum, lut, outb,
                   hist_sh, lut_sh):
    c = lax.axis_index("c")
    s = lax.axis_index("s")
    iota = lax.iota(jnp.int32, LANES)
    lane_base = iota * NBINS
    ones = jnp.ones((LANES,), jnp.float32)
    zeros = jnp.zeros((LANES,), jnp.float32)

    def p1_body(ch, _):
        base = (c * NCH_PER_CORE + ch) * NPIX + s * CHUNK
        pltpu.sync_copy(img.at[pl.ds(base, CHUNK)], pix)

        @plsc.parallel_loop(0, NBINS // LANES, 1, unroll=2)
        def _(j):
            for r in range(NSUB):
                hist[r, pl.ds(j * LANES, LANES)] = zeros

        @plsc.parallel_loop(0, CHUNK // LANES, 1, unroll=8)
        def _(i):
            v = pix[pl.ds(i * LANES, LANES)]
            plsc.addupdate_scatter(hist, [iota, v], ones)

        @plsc.parallel_loop(0, NBINS // LANES, 1, unroll=2)
        def _(j):
            acc = hist[0, pl.ds(j * LANES, LANES)]
            for r in range(1, LANES):
                acc = acc + hist[r, pl.ds(j * LANES, LANES)]
            histred[pl.ds(j * LANES, LANES)] = acc

        pltpu.sync_copy(histred, hist_sh.at[ch, s])
        return 0
    lax.fori_loop(0, NCH_PER_CORE, p1_body, 0)

    plsc.subcore_barrier()

    def make_lut(chv):
        pltpu.sync_copy(hist_sh.at[chv], hist)

        def rbody(j, _):
            acc = hist[0, pl.ds(j * LANES, LANES)]
            for r in range(1, NSUB):
                acc = acc + hist[r, pl.ds(j * LANES, LANES)]
            histred[pl.ds(j * LANES, LANES)] = acc
            return 0
        lax.fori_loop(0, NBINS // LANES, rbody, 0)

        def cbody(j, carry):
            cacc, li = carry
            x = histred[pl.ds(j * LANES, LANES)]
            cs = plsc.cumsum(x) + cacc
            cum[pl.ds(j * LANES, LANES)] = cs
            gidx = iota + j * LANES
            ljm = jnp.max(jnp.where(x > 0.0, gidx, -1))
            return (jnp.max(cs), jnp.maximum(li, ljm))
        total, li = lax.fori_loop(
            0, NBINS // LANES, cbody, (jnp.float32(0.0), jnp.int32(-1)))

        def hbody(j, acc):
            x = histred[pl.ds(j * LANES, LANES)]
            gidx = iota + j * LANES
            return acc + jnp.sum(jnp.where(gidx == li, x, 0.0))
        hist_last = lax.fori_loop(0, NBINS // LANES, hbody, jnp.float32(0.0))

        num_v = jnp.full((LANES,), total - hist_last, jnp.float32)
        step = _floorf(num_v / 255.0)
        half = _floorf(step * 0.5)
        div = jnp.maximum(step, 1.0)
        ident = step <= 0.0

        def lbody(j, _):
            cs = cum[pl.ds(j * LANES, LANES)]
            val = jnp.clip(_floorf((cs + half) / div), 0.0, 255.0)
            gidx = iota + j * LANES
            val = jnp.where(ident, (gidx + 1).astype(jnp.float32), val)
            plsc.store_scatter(lut, [gidx + 1], val, mask=gidx < NBINS - 1)
            return 0
        lax.fori_loop(0, NBINS // LANES, lbody, 0)
        v0 = lut[pl.ds(0, LANES)]
        lut[pl.ds(0, LANES)] = jnp.where(iota == 0, 0.0, v0)

        pltpu.sync_copy(lut, lut_sh.at[chv])

    for rep in range(2):
        chv = s + NSUB * rep

        @pl.when(chv < NCH_PER_CORE)
        def _(chv=chv):
            make_lut(chv)

    plsc.subcore_barrier()

    def p2_body(ch, _):
        base = (c * NCH_PER_CORE + ch) * NPIX + s * CHUNK
        pltpu.sync_copy(lut_sh.at[ch], lut)
        pltpu.sync_copy(img.at[pl.ds(base, CHUNK)], pix)

        @plsc.parallel_loop(0, CHUNK // LANES, 1, unroll=8)
        def _(i):
            sl = pl.ds(i * LANES, LANES)
            outb[sl] = plsc.load_gather(lut, [pix[sl]])

        pltpu.sync_copy(outb, out.at[pl.ds(base, CHUNK)])
        return 0
    lax.fori_loop(0, NCH_PER_CORE, p2_body, 0)


@jax.jit
def kernel(image):
    B, C, H, W = image.shape
    n = B * C * H * W
    flat = image.reshape(n)

    mesh = plsc.VectorSubcoreMesh(
        core_axis_name="c", subcore_axis_name="s",
        num_cores=NCORES, num_subcores=NSUB)
    eq = pl.kernel(
        _equalize_body,
        out_type=jax.ShapeDtypeStruct((n,), jnp.float32),
        mesh=mesh,
        compiler_params=pltpu.CompilerParams(
            use_tc_tiling_on_sc=False, needs_layout_passes=False),
        scratch_types=[
            pltpu.VMEM((CHUNK,), jnp.int32),
            pltpu.VMEM((NSUB, NBINS), jnp.float32),
            pltpu.VMEM((NBINS,), jnp.float32),
            pltpu.VMEM((NBINS,), jnp.float32),
            pltpu.VMEM((NBINS,), jnp.float32),
            pltpu.VMEM((CHUNK,), jnp.float32),
            pltpu.VMEM_SHARED((NCH_PER_CORE, NSUB, NBINS), jnp.float32),
            pltpu.VMEM_SHARED((NCH_PER_CORE, NBINS), jnp.float32),
        ],
    )
    return eq(flat).reshape(B, C, H, W)


# speedup vs baseline: 736.8316x; 1.3884x over previous
import jax
import jax.numpy as jnp
from jax import lax
from jax.experimental import pallas as pl
from jax.experimental.pallas import tpu as pltpu
from jax.experimental.pallas import tpu_sc as plsc

NCORES = 2
NSUB = 16
LANES = 16
NPIX = 512 * 512
CHUNK = NPIX // NSUB
NCH = 24
NBINS = 256


def _floorf(x):
    return x.astype(jnp.int32).astype(jnp.float32)


def _equalize_body(img, out, pix_a, pix_b, out_a, out_b, hist24, part,
                   histred, cum, lutall, hist_sh, lut_sh,
                   sem_ia, sem_ib, sem_oa, sem_ob):
    c = lax.axis_index("c")
    s = lax.axis_index("s")
    iota = lax.iota(jnp.int32, LANES)
    ones = jnp.ones((LANES,), jnp.float32)
    zeros = jnp.zeros((LANES,), jnp.float32)

    def in_slice(ch):
        base = (c * NCH + ch) * NPIX + s * CHUNK
        return img.at[pl.ds(base, CHUNK)]

    def out_slice(ch):
        base = (c * NCH + ch) * NPIX + s * CHUNK
        return out.at[pl.ds(base, CHUNK)]

    @plsc.parallel_loop(0, NCH * NBINS // LANES, 1, unroll=4)
    def _(j):
        r = j >> 4
        col = (j & 15) * LANES
        hist24[r, pl.ds(col, LANES)] = zeros

    def scatter_chunk(pix, ch):
        chv = jnp.full((LANES,), ch, jnp.int32)

        @plsc.parallel_loop(0, CHUNK // LANES, 1, unroll=8)
        def _(i):
            v = pix[pl.ds(i * LANES, LANES)]
            plsc.addupdate_scatter(hist24, [chv, v], ones)

    pltpu.async_copy(in_slice(0), pix_a, sem_ia)

    def p1_body(j, _):
        ch_a = 2 * j
        ch_b = 2 * j + 1
        pltpu.async_copy(in_slice(ch_b), pix_b, sem_ib)
        pltpu.make_async_copy(in_slice(ch_a), pix_a, sem_ia).wait()
        scatter_chunk(pix_a, ch_a)

        @pl.when(ch_a + 2 < NCH)
        def _():
            pltpu.async_copy(in_slice(ch_a + 2), pix_a, sem_ia)
        pltpu.make_async_copy(in_slice(ch_b), pix_b, sem_ib).wait()
        scatter_chunk(pix_b, ch_b)
        return 0
    lax.fori_loop(0, NCH // 2, p1_body, 0)

    pltpu.sync_copy(hist24, hist_sh.at[s])
    plsc.subcore_barrier()

    def make_lut(chv):
        pltpu.sync_copy(hist_sh.at[:, chv], part)

        @plsc.parallel_loop(0, NBINS // LANES, 1, unroll=2)
        def _(j):
            acc = part[0, pl.ds(j * LANES, LANES)]
            for r in range(1, NSUB):
                acc = acc + part[r, pl.ds(j * LANES, LANES)]
            histred[pl.ds(j * LANES, LANES)] = acc

        def cbody(j, carry):
            cacc, li = carry
            x = histred[pl.ds(j * LANES, LANES)]
            cs = plsc.cumsum(x) + cacc
            cum[pl.ds(j * LANES, LANES)] = cs
            gidx = iota + j * LANES
            ljm = jnp.max(jnp.where(x > 0.0, gidx, -1))
            return (jnp.max(cs), jnp.maximum(li, ljm))
        total, li = lax.fori_loop(
            0, NBINS // LANES, cbody, (jnp.float32(0.0), jnp.int32(-1)))

        def hbody(j, acc):
            x = histred[pl.ds(j * LANES, LANES)]
            gidx = iota + j * LANES
            return acc + jnp.sum(jnp.where(gidx == li, x, 0.0))
        hist_last = lax.fori_loop(0, NBINS // LANES, hbody, jnp.float32(0.0))

        num_v = jnp.full((LANES,), total - hist_last, jnp.float32)
        step = _floorf(num_v / 255.0)
        half = _floorf(step * 0.5)
        div = jnp.maximum(step, 1.0)
        ident = step <= 0.0
        chs = jnp.full((LANES,), chv, jnp.int32)

        def lbody(j, _):
            cs = cum[pl.ds(j * LANES, LANES)]
            val = jnp.clip(_floorf((cs + half) / div), 0.0, 255.0)
            gidx = iota + j * LANES
            val = jnp.where(ident, (gidx + 1).astype(jnp.float32), val)
            plsc.store_scatter(
                lutall, [chs, gidx + 1], val, mask=gidx < NBINS - 1)
            return 0
        lax.fori_loop(0, NBINS // LANES, lbody, 0)
        v0 = lutall[chv, pl.ds(0, LANES)]
        lutall[chv, pl.ds(0, LANES)] = jnp.where(iota == 0, 0.0, v0)

        pltpu.sync_copy(lutall.at[chv], lut_sh.at[chv])

    for rep in range(2):
        chx = s + NSUB * rep

        @pl.when(chx < NCH)
        def _(chx=chx):
            make_lut(chx)

    plsc.subcore_barrier()

    pltpu.sync_copy(lut_sh, lutall)

    def gather_chunk(pix, outb, ch):
        chv = jnp.full((LANES,), ch, jnp.int32)

        @plsc.parallel_loop(0, CHUNK // LANES, 1, unroll=8)
        def _(i):
            sl = pl.ds(i * LANES, LANES)
            outb[sl] = plsc.load_gather(lutall, [chv, pix[sl]])

    pltpu.async_copy(in_slice(0), pix_a, sem_ia)

    def p2_body(j, _):
        ch_a = 2 * j
        ch_b = 2 * j + 1
        pltpu.async_copy(in_slice(ch_b), pix_b, sem_ib)
        pltpu.make_async_copy(in_slice(ch_a), pix_a, sem_ia).wait()

        @pl.when(j > 0)
        def _():
            pltpu.make_async_copy(out_a, out_slice(ch_a - 2), sem_oa).wait()
        gather_chunk(pix_a, out_a, ch_a)
        pltpu.async_copy(out_a, out_slice(ch_a), sem_oa)

        @pl.when(ch_a + 2 < NCH)
        def _():
            pltpu.async_copy(in_slice(ch_a + 2), pix_a, sem_ia)
        pltpu.make_async_copy(in_slice(ch_b), pix_b, sem_ib).wait()

        @pl.when(j > 0)
        def _():
            pltpu.make_async_copy(out_b, out_slice(ch_b - 2), sem_ob).wait()
        gather_chunk(pix_b, out_b, ch_b)
        pltpu.async_copy(out_b, out_slice(ch_b), sem_ob)
        return 0
    lax.fori_loop(0, NCH // 2, p2_body, 0)

    pltpu.make_async_copy(out_a, out_slice(NCH - 2), sem_oa).wait()
    pltpu.make_async_copy(out_b, out_slice(NCH - 1), sem_ob).wait()


@jax.jit
def kernel(image):
    B, C, H, W = image.shape
    n = B * C * H * W
    flat = image.reshape(n)

    mesh = plsc.VectorSubcoreMesh(
        core_axis_name="c", subcore_axis_name="s",
        num_cores=NCORES, num_subcores=NSUB)
    eq = pl.kernel(
        _equalize_body,
        out_type=jax.ShapeDtypeStruct((n,), jnp.float32),
        mesh=mesh,
        compiler_params=pltpu.CompilerParams(
            use_tc_tiling_on_sc=False, needs_layout_passes=False),
        scratch_types=[
            pltpu.VMEM((CHUNK,), jnp.int32),
            pltpu.VMEM((CHUNK,), jnp.int32),
            pltpu.VMEM((CHUNK,), jnp.float32),
            pltpu.VMEM((CHUNK,), jnp.float32),
            pltpu.VMEM((NCH, NBINS), jnp.float32),
            pltpu.VMEM((NSUB, NBINS), jnp.float32),
            pltpu.VMEM((NBINS,), jnp.float32),
            pltpu.VMEM((NBINS,), jnp.float32),
            pltpu.VMEM((NCH, NBINS), jnp.float32),
            pltpu.VMEM_SHARED((NSUB, NCH, NBINS), jnp.float32),
            pltpu.VMEM_SHARED((NCH, NBINS), jnp.float32),
            pltpu.SemaphoreType.DMA,
            pltpu.SemaphoreType.DMA,
            pltpu.SemaphoreType.DMA,
            pltpu.SemaphoreType.DMA,
        ],
    )
    return eq(flat).reshape(B, C, H, W)


# speedup vs baseline: 1318.6756x; 1.7897x over previous
import jax
import jax.numpy as jnp
from jax import lax
from jax.experimental import pallas as pl
from jax.experimental.pallas import tpu as pltpu
from jax.experimental.pallas import tpu_sc as plsc

NCORES = 2
NSUB = 16
LANES = 16
NPIX = 512 * 512
CHUNK = NPIX // NSUB
NCH = 24
NBINS = 256
ROWS = 512 // NSUB


def _floorf(x):
    return x.astype(jnp.int32).astype(jnp.float32)


def _equalize_body(img, out, pix_a, pix_b, out_a, out_b, hist24, part,
                   histred, cum, lutall, hist_sh, lut_sh,
                   sem_ia, sem_ib, sem_oa, sem_ob):
    c = lax.axis_index("c")
    s = lax.axis_index("s")
    iota = lax.iota(jnp.int32, LANES)
    ones = jnp.ones((LANES,), jnp.float32)
    zeros = jnp.zeros((LANES,), jnp.float32)

    def in_slice(ch):
        return img.at[c * NCH + ch, pl.ds(s * ROWS, ROWS)]

    def out_slice(ch):
        return out.at[c * NCH + ch, pl.ds(s * ROWS, ROWS)]

    @plsc.parallel_loop(0, NCH * NBINS // LANES, 1, unroll=4)
    def _(j):
        r = j >> 4
        col = (j & 15) * LANES
        hist24[r, pl.ds(col, LANES)] = zeros

    def scatter_chunk(pix, ch):
        chv = jnp.full((LANES,), ch, jnp.int32)

        @plsc.parallel_loop(0, CHUNK // LANES, 1, unroll=8)
        def _(i):
            v = pix[i >> 5, pl.ds((i & 31) * LANES, LANES)]
            plsc.addupdate_scatter(hist24, [chv, v], ones)

    pltpu.async_copy(in_slice(0), pix_a, sem_ia)

    def p1_body(j, _):
        ch_a = 2 * j
        ch_b = 2 * j + 1
        pltpu.async_copy(in_slice(ch_b), pix_b, sem_ib)
        pltpu.make_async_copy(in_slice(ch_a), pix_a, sem_ia).wait()
        scatter_chunk(pix_a, ch_a)

        @pl.when(ch_a + 2 < NCH)
        def _():
            pltpu.async_copy(in_slice(ch_a + 2), pix_a, sem_ia)
        pltpu.make_async_copy(in_slice(ch_b), pix_b, sem_ib).wait()
        scatter_chunk(pix_b, ch_b)
        return 0
    lax.fori_loop(0, NCH // 2, p1_body, 0)

    pltpu.sync_copy(hist24, hist_sh.at[s])
    plsc.subcore_barrier()

    def make_lut(chv):
        pltpu.sync_copy(hist_sh.at[:, chv], part)

        @plsc.parallel_loop(0, NBINS // LANES, 1, unroll=2)
        def _(j):
            acc = part[0, pl.ds(j * LANES, LANES)]
            for r in range(1, NSUB):
                acc = acc + part[r, pl.ds(j * LANES, LANES)]
            histred[pl.ds(j * LANES, LANES)] = acc

        def cbody(j, carry):
            cacc, li = carry
            x = histred[pl.ds(j * LANES, LANES)]
            cs = plsc.cumsum(x) + cacc
            cum[pl.ds(j * LANES, LANES)] = cs
            gidx = iota + j * LANES
            ljm = jnp.max(jnp.where(x > 0.0, gidx, -1))
            return (jnp.max(cs), jnp.maximum(li, ljm))
        total, li = lax.fori_loop(
            0, NBINS // LANES, cbody, (jnp.float32(0.0), jnp.int32(-1)))

        def hbody(j, acc):
            x = histred[pl.ds(j * LANES, LANES)]
            gidx = iota + j * LANES
            return acc + jnp.sum(jnp.where(gidx == li, x, 0.0))
        hist_last = lax.fori_loop(0, NBINS // LANES, hbody, jnp.float32(0.0))

        num_v = jnp.full((LANES,), total - hist_last, jnp.float32)
        step = _floorf(num_v / 255.0)
        half = _floorf(step * 0.5)
        div = jnp.maximum(step, 1.0)
        ident = step <= 0.0
        chs = jnp.full((LANES,), chv, jnp.int32)

        def lbody(j, _):
            cs = cum[pl.ds(j * LANES, LANES)]
            val = jnp.clip(_floorf((cs + half) / div), 0.0, 255.0)
            gidx = iota + j * LANES
            val = jnp.where(ident, (gidx + 1).astype(jnp.float32), val)
            plsc.store_scatter(
                lutall, [chs, gidx + 1], val, mask=gidx < NBINS - 1)
            return 0
        lax.fori_loop(0, NBINS // LANES, lbody, 0)
        v0 = lutall[chv, pl.ds(0, LANES)]
        lutall[chv, pl.ds(0, LANES)] = jnp.where(iota == 0, 0.0, v0)

        pltpu.sync_copy(lutall.at[chv], lut_sh.at[chv])

    for rep in range(2):
        chx = s + NSUB * rep

        @pl.when(chx < NCH)
        def _(chx=chx):
            make_lut(chx)

    plsc.subcore_barrier()

    pltpu.sync_copy(lut_sh, lutall)

    def gather_chunk(pix, outb, ch):
        chv = jnp.full((LANES,), ch, jnp.int32)

        @plsc.parallel_loop(0, CHUNK // LANES, 1, unroll=8)
        def _(i):
            r = i >> 5
            sl = pl.ds((i & 31) * LANES, LANES)
            outb[r, sl] = plsc.load_gather(lutall, [chv, pix[r, sl]])

    pltpu.async_copy(in_slice(0), pix_a, sem_ia)

    def p2_body(j, _):
        ch_a = 2 * j
        ch_b = 2 * j + 1
        pltpu.async_copy(in_slice(ch_b), pix_b, sem_ib)
        pltpu.make_async_copy(in_slice(ch_a), pix_a, sem_ia).wait()

        @pl.when(j > 0)
        def _():
            pltpu.make_async_copy(out_a, out_slice(ch_a - 2), sem_oa).wait()
        gather_chunk(pix_a, out_a, ch_a)
        pltpu.async_copy(out_a, out_slice(ch_a), sem_oa)

        @pl.when(ch_a + 2 < NCH)
        def _():
            pltpu.async_copy(in_slice(ch_a + 2), pix_a, sem_ia)
        pltpu.make_async_copy(in_slice(ch_b), pix_b, sem_ib).wait()

        @pl.when(j > 0)
        def _():
            pltpu.make_async_copy(out_b, out_slice(ch_b - 2), sem_ob).wait()
        gather_chunk(pix_b, out_b, ch_b)
        pltpu.async_copy(out_b, out_slice(ch_b), sem_ob)
        return 0
    lax.fori_loop(0, NCH // 2, p2_body, 0)

    pltpu.make_async_copy(out_a, out_slice(NCH - 2), sem_oa).wait()
    pltpu.make_async_copy(out_b, out_slice(NCH - 1), sem_ob).wait()


@jax.jit
def kernel(image):
    B, C, H, W = image.shape
    flat = image.reshape(B * C, H, W)

    mesh = plsc.VectorSubcoreMesh(
        core_axis_name="c", subcore_axis_name="s",
        num_cores=NCORES, num_subcores=NSUB)
    eq = pl.kernel(
        _equalize_body,
        out_type=jax.ShapeDtypeStruct((B * C, H, W), jnp.float32),
        mesh=mesh,
        compiler_params=pltpu.CompilerParams(
            use_tc_tiling_on_sc=True, needs_layout_passes=False),
        scratch_types=[
            pltpu.VMEM((ROWS, 512), jnp.int32),
            pltpu.VMEM((ROWS, 512), jnp.int32),
            pltpu.VMEM((ROWS, 512), jnp.float32),
            pltpu.VMEM((ROWS, 512), jnp.float32),
            pltpu.VMEM((NCH, NBINS), jnp.float32),
            pltpu.VMEM((NSUB, NBINS), jnp.float32),
            pltpu.VMEM((NBINS,), jnp.float32),
            pltpu.VMEM((NBINS,), jnp.float32),
            pltpu.VMEM((NCH, NBINS), jnp.float32),
            pltpu.VMEM_SHARED((NSUB, NCH, NBINS), jnp.float32),
            pltpu.VMEM_SHARED((NCH, NBINS), jnp.float32),
            pltpu.SemaphoreType.DMA,
            pltpu.SemaphoreType.DMA,
            pltpu.SemaphoreType.DMA,
            pltpu.SemaphoreType.DMA,
        ],
    )
    return eq(flat).reshape(B, C, H, W)
